# baseline (speedup 1.0000x reference)
import jax
import jax.numpy as jnp
from jax import lax
from jax.experimental import pallas as pl
from jax.experimental.pallas import tpu as pltpu
from jax.experimental.pallas import tpu_sc as plsc

N = 10000
E = 320000
D = 128
H = 128
C = 10
G = 64

NCORE = 2
NSUB = 16

B = 32
WD = 128
NCD = 320
EPAD = NCORE * NSUB * NCD * B
NROWS = 10240
STRIPE = NROWS // NSUB
BLK = 1024
GRID = NROWS // BLK
NBUF = 8
TWO_N = 2 * NBUF


def _deg_body(dst_hbm, ones_hbm, zeros_hbm, out_hbm, ibuf, ones_v, sem_i, acc):
    cid = lax.axis_index("c")
    sid = lax.axis_index("s")
    wid = sid * NCORE + cid
    base = wid * NCD
    pltpu.sync_copy(zeros_hbm, acc.at[pl.ds(sid * STRIPE, STRIPE)])
    pltpu.sync_copy(ones_hbm, ones_v)
    plsc.subcore_barrier()

    def idx_start(c, slot):
        pltpu.async_copy(dst_hbm.at[pl.ds(base + c, 1)],
                         ibuf.at[slot], sem_i.at[slot])

    def idx_wait(slot):
        pltpu.make_async_copy(dst_hbm.at[pl.ds(0, 1)],
                              ibuf.at[slot], sem_i.at[slot]).wait()

    for b in range(TWO_N):
        idx_start(b, b)

    @pl.loop(0, NCD, step=TWO_N)
    def _(j):
        for b in range(TWO_N):
            c = j + b
            idx_wait(b)
            pltpu.sync_copy(ones_v, acc.at[ibuf.at[b, 0]], add=True)

            @pl.when(c + TWO_N < NCD)
            def _():
                idx_start(c + TWO_N, b)

    plsc.subcore_barrier()
    pltpu.sync_copy(acc.at[pl.ds(sid * STRIPE, STRIPE)],
                    out_hbm.at[cid, pl.ds(sid * STRIPE, STRIPE)])


def _deg_pass(dst_idx, ones128, zeros128):
    k = pl.kernel(
        _deg_body,
        out_type=jax.ShapeDtypeStruct((NCORE, NROWS, WD), jnp.float32),
        mesh=plsc.VectorSubcoreMesh(
            core_axis_name="c", subcore_axis_name="s",
            num_cores=NCORE, num_subcores=NSUB),
        scratch_types=[
            pltpu.VMEM((TWO_N, 1, B), jnp.int32),
            pltpu.VMEM((B, WD), jnp.float32),
            pltpu.SemaphoreType.DMA((TWO_N,)),
            pltpu.VMEM_SHARED((NROWS, WD), jnp.float32),
        ],
    )
    return k(dst_idx, ones128, zeros128)


def _spmm_body(t_hbm, ei_hbm, zeros_hbm, out_hbm,
               ibuf, gbuf, sem_i, sem_g, acc):
    cid = lax.axis_index("c")
    sid = lax.axis_index("s")
    wid = sid * NCORE + cid
    base = wid * NCD * 2
    pltpu.sync_copy(zeros_hbm, acc.at[pl.ds(sid * STRIPE, STRIPE)])
    plsc.subcore_barrier()

    def idx_start(c, slot):
        pltpu.async_copy(ei_hbm.at[pl.ds(base + 2 * c, 2)],
                         ibuf.at[slot], sem_i.at[slot])

    def idx_wait(slot):
        pltpu.make_async_copy(ei_hbm.at[pl.ds(0, 2)],
                              ibuf.at[slot], sem_i.at[slot]).wait()

    def g_start(islot, gslot):
        pltpu.async_copy(t_hbm.at[ibuf.at[islot, 0]],
                         gbuf.at[gslot], sem_g.at[gslot])

    def g_wait(gslot):
        pltpu.make_async_copy(t_hbm.at[pl.ds(0, B)],
                              gbuf.at[gslot], sem_g.at[gslot]).wait()

    for b in range(TWO_N):
        idx_start(b, b)
    for b in range(NBUF):
        idx_wait(b)
        g_start(b, b)

    @pl.loop(0, NCD, step=TWO_N)
    def _(j):
        for b in range(TWO_N):
            c = j + b
            gs = b % NBUF
            is2 = (b + NBUF) % TWO_N
            g_wait(gs)
            pltpu.sync_copy(gbuf.at[gs], acc.at[ibuf.at[b, 1]], add=True)

            @pl.when(c + TWO_N < NCD)
            def _():
                idx_start(c + TWO_N, b)

            @pl.when(c + NBUF < NCD)
            def _():
                idx_wait(is2)
                g_start(is2, gs)

    plsc.subcore_barrier()
    pltpu.sync_copy(acc.at[pl.ds(sid * STRIPE, STRIPE)],
                    out_hbm.at[cid, pl.ds(sid * STRIPE, STRIPE)])


def _spmm_pass(t, ei, zeros128):
    k = pl.kernel(
        _spmm_body,
        out_type=jax.ShapeDtypeStruct((NCORE, NROWS, H), jnp.float32),
        mesh=plsc.VectorSubcoreMesh(
            core_axis_name="c", subcore_axis_name="s",
            num_cores=NCORE, num_subcores=NSUB),
        scratch_types=[
            pltpu.VMEM((TWO_N, 2, B), jnp.int32),
            pltpu.VMEM((NBUF, B, H), jnp.float32),
            pltpu.SemaphoreType.DMA((TWO_N,)),
            pltpu.SemaphoreType.DMA((NBUF,)),
            pltpu.VMEM_SHARED((NROWS, H), jnp.float32),
        ],
    )
    return k(t, ei, zeros128)


def _tc_first_body(x_ref, d0_ref, d1_ref, w_ref, t_ref, dinv_ref):
    deg = d0_ref[...] + d1_ref[...] + 1.0
    dinv = lax.rsqrt(deg)
    dinv_ref[...] = dinv[:, 0:16]
    t_ref[...] = dinv[:, 0:1] * jnp.dot(
        x_ref[...], w_ref[...], preferred_element_type=jnp.float32)


def _tc_first(x, degp, w):
    return pl.pallas_call(
        _tc_first_body,
        grid=(GRID,),
        in_specs=[
            pl.BlockSpec((BLK, D), lambda i: (i, 0)),
            pl.BlockSpec((BLK, WD), lambda i: (i, 0)),
            pl.BlockSpec((BLK, WD), lambda i: (i, 0)),
            pl.BlockSpec((D, H), lambda i: (0, 0)),
        ],
        out_specs=[
            pl.BlockSpec((BLK, H), lambda i: (i, 0)),
            pl.BlockSpec((BLK, 16), lambda i: (i, 0)),
        ],
        out_shape=[
            jax.ShapeDtypeStruct((NROWS, H), jnp.float32),
            jax.ShapeDtypeStruct((NROWS, 16), jnp.float32),
        ],
    )(x, degp[0], degp[1], w)


def _tc_mid_body(p0_ref, p1_ref, t_ref, dinv_ref, b_ref, w_ref, o_ref):
    dinv = dinv_ref[:, 0:1]
    s = p0_ref[...] + p1_ref[...] + t_ref[...]
    h = jnp.maximum(dinv * s + b_ref[...], 0.0)
    o_ref[...] = dinv * jnp.dot(
        h, w_ref[...], preferred_element_type=jnp.float32)


def _tc_mid(p, t, dinv, bias, w):
    return pl.pallas_call(
        _tc_mid_body,
        grid=(GRID,),
        in_specs=[
            pl.BlockSpec((BLK, H), lambda i: (i, 0)),
            pl.BlockSpec((BLK, H), lambda i: (i, 0)),
            pl.BlockSpec((BLK, H), lambda i: (i, 0)),
            pl.BlockSpec((BLK, 16), lambda i: (i, 0)),
            pl.BlockSpec((1, H), lambda i: (0, 0)),
            pl.BlockSpec((H, H), lambda i: (0, 0)),
        ],
        out_specs=pl.BlockSpec((BLK, H), lambda i: (i, 0)),
        out_shape=jax.ShapeDtypeStruct((NROWS, H), jnp.float32),
    )(p[0], p[1], t, dinv, bias, w)


def _tc_final_body(p0_ref, p1_ref, t_ref, dinv_ref, b_ref, batch_ref,
                   wl_ref, bl_ref, o_ref, pool_acc, cnt_acc):
    i = pl.program_id(0)

    @pl.when(i == 0)
    def _():
        pool_acc[...] = jnp.zeros_like(pool_acc)
        cnt_acc[...] = jnp.zeros_like(cnt_acc)

    dinv = dinv_ref[:, 0:1]
    h = dinv * (p0_ref[...] + p1_ref[...] + t_ref[...]) + b_ref[...]
    gid = batch_ref[...]
    onehot = jnp.where(
        gid == lax.broadcasted_iota(jnp.int32, (BLK, G), 1), 1.0, 0.0)
    pool_acc[...] += lax.dot_general(
        onehot, h, (((0,), (0,)), ((), ())),
        preferred_element_type=jnp.float32)
    cnt_acc[...] += lax.dot_general(
        onehot, jnp.ones((BLK, H), jnp.float32), (((0,), (0,)), ((), ())),
        preferred_element_type=jnp.float32)

    @pl.when(i == GRID - 1)
    def _():
        pooled = pool_acc[...] / jnp.maximum(cnt_acc[...], 1.0)
        o_ref[...] = jnp.dot(
            pooled, wl_ref[...], preferred_element_type=jnp.float32
        ) + bl_ref[...]


def _tc_final(p, t, dinv, bias, batch2d, wl, bl):
    return pl.pallas_call(
        _tc_final_body,
        grid=(GRID,),
        in_specs=[
            pl.BlockSpec((BLK, H), lambda i: (i, 0)),
            pl.BlockSpec((BLK, H), lambda i: (i, 0)),
            pl.BlockSpec((BLK, H), lambda i: (i, 0)),
            pl.BlockSpec((BLK, 16), lambda i: (i, 0)),
            pl.BlockSpec((1, H), lambda i: (0, 0)),
            pl.BlockSpec((BLK, 1), lambda i: (i, 0)),
            pl.BlockSpec((H, C), lambda i: (0, 0)),
            pl.BlockSpec((1, C), lambda i: (0, 0)),
        ],
        out_specs=pl.BlockSpec((G, C), lambda i: (0, 0)),
        out_shape=jax.ShapeDtypeStruct((G, C), jnp.float32),
        scratch_shapes=[
            pltpu.VMEM((G, H), jnp.float32),
            pltpu.VMEM((G, H), jnp.float32),
        ],
    )(p[0], p[1], t, dinv, bias, batch2d, wl, bl)


@jax.jit
def kernel(x, edge_index, batch, W1, b1, W2, b2, W3, b3, Wl, bl):
    pad_e = EPAD - E
    pad_ids = (jnp.arange(pad_e, dtype=jnp.int32) % (NROWS - N)) + N
    src = jnp.concatenate([edge_index[0], pad_ids]).reshape(-1, B)
    dst = jnp.concatenate([edge_index[1], pad_ids]).reshape(-1, B)
    nw = NCORE * NSUB
    ei = jnp.stack([src.reshape(nw, NCD, B),
                    dst.reshape(nw, NCD, B)], axis=2).reshape(-1, B)
    xp = jnp.concatenate(
        [x, jnp.zeros((NROWS - N, D), jnp.float32)], axis=0)
    batch2d = jnp.concatenate(
        [batch, jnp.full((NROWS - N,), G, jnp.int32)]).reshape(NROWS, 1)
    ones_deg = jnp.ones((B, WD), jnp.float32)
    zeros_deg = jnp.zeros((STRIPE, WD), jnp.float32)
    zeros128 = jnp.zeros((STRIPE, H), jnp.float32)
    b1r = b1.reshape(1, H)
    b2r = b2.reshape(1, H)
    b3r = b3.reshape(1, H)
    blr = bl.reshape(1, C)

    degp = _deg_pass(dst, ones_deg, zeros_deg)
    t1, dinv = _tc_first(xp, degp, W1)
    p1 = _spmm_pass(t1, ei, zeros128)
    t2 = _tc_mid(p1, t1, dinv, b1r, W2)
    p2 = _spmm_pass(t2, ei, zeros128)
    t3 = _tc_mid(p2, t2, dinv, b2r, W3)
    p3 = _spmm_pass(t3, ei, zeros128)
    return _tc_final(p3, t3, dinv, b3r, batch2d, Wl, blr)


# speedup vs baseline: 1.1147x; 1.1147x over previous
import jax
import jax.numpy as jnp
from jax import lax
from jax.experimental import pallas as pl
from jax.experimental.pallas import tpu as pltpu
from jax.experimental.pallas import tpu_sc as plsc

N = 10000
E = 320000
D = 128
H = 128
C = 10
G = 64

NCORE = 2
NSUB = 16

B = 64
WD = 128
NCD = 160
EPAD = NCORE * NSUB * NCD * B
NROWS = 10240
STRIPE = NROWS // NSUB
BLK = 1024
GRID = NROWS // BLK
NBUF = 4
TWO_N = 2 * NBUF


def _deg_body(dst_hbm, ones_hbm, zeros_hbm, out_hbm, ibuf, ones_v, sem_i, acc):
    cid = lax.axis_index("c")
    sid = lax.axis_index("s")
    wid = sid * NCORE + cid
    base = wid * NCD
    pltpu.sync_copy(zeros_hbm, acc.at[pl.ds(sid * STRIPE, STRIPE)])
    pltpu.sync_copy(ones_hbm, ones_v)
    plsc.subcore_barrier()

    def idx_start(c, slot):
        pltpu.async_copy(dst_hbm.at[pl.ds(base + c, 1)],
                         ibuf.at[slot], sem_i.at[slot])

    def idx_wait(slot):
        pltpu.make_async_copy(dst_hbm.at[pl.ds(0, 1)],
                              ibuf.at[slot], sem_i.at[slot]).wait()

    for b in range(TWO_N):
        idx_start(b, b)

    @pl.loop(0, NCD, step=TWO_N)
    def _(j):
        for b in range(TWO_N):
            c = j + b
            idx_wait(b)
            pltpu.sync_copy(ones_v, acc.at[ibuf.at[b, 0]], add=True)

            @pl.when(c + TWO_N < NCD)
            def _():
                idx_start(c + TWO_N, b)

    plsc.subcore_barrier()
    pltpu.sync_copy(acc.at[pl.ds(sid * STRIPE, STRIPE)],
                    out_hbm.at[cid, pl.ds(sid * STRIPE, STRIPE)])


def _deg_pass(dst_idx, ones128, zeros128):
    k = pl.kernel(
        _deg_body,
        out_type=jax.ShapeDtypeStruct((NCORE, NROWS, WD), jnp.float32),
        mesh=plsc.VectorSubcoreMesh(
            core_axis_name="c", subcore_axis_name="s",
            num_cores=NCORE, num_subcores=NSUB),
        scratch_types=[
            pltpu.VMEM((TWO_N, 1, B), jnp.int32),
            pltpu.VMEM((B, WD), jnp.float32),
            pltpu.SemaphoreType.DMA((TWO_N,)),
            pltpu.VMEM_SHARED((NROWS, WD), jnp.float32),
        ],
    )
    return k(dst_idx, ones128, zeros128)


def _spmm_body(t_hbm, ei_hbm, zeros_hbm, out_hbm,
               ibuf, gbuf, sem_i, sem_g, acc):
    cid = lax.axis_index("c")
    sid = lax.axis_index("s")
    wid = sid * NCORE + cid
    base = wid * NCD * 2
    pltpu.sync_copy(zeros_hbm, acc.at[pl.ds(sid * STRIPE, STRIPE)])
    plsc.subcore_barrier()

    def idx_start(c, slot):
        pltpu.async_copy(ei_hbm.at[pl.ds(base + 2 * c, 2)],
                         ibuf.at[slot], sem_i.at[slot])

    def idx_wait(slot):
        pltpu.make_async_copy(ei_hbm.at[pl.ds(0, 2)],
                              ibuf.at[slot], sem_i.at[slot]).wait()

    def g_start(islot, gslot):
        pltpu.async_copy(t_hbm.at[ibuf.at[islot, 0]],
                         gbuf.at[gslot], sem_g.at[gslot])

    def g_wait(gslot):
        pltpu.make_async_copy(t_hbm.at[pl.ds(0, B)],
                              gbuf.at[gslot], sem_g.at[gslot]).wait()

    for b in range(TWO_N):
        idx_start(b, b)
    for b in range(NBUF):
        idx_wait(b)
        g_start(b, b)

    @pl.loop(0, NCD, step=TWO_N)
    def _(j):
        for b in range(TWO_N):
            c = j + b
            gs = b % NBUF
            is2 = (b + NBUF) % TWO_N
            g_wait(gs)
            pltpu.sync_copy(gbuf.at[gs], acc.at[ibuf.at[b, 1]], add=True)

            @pl.when(c + TWO_N < NCD)
            def _():
                idx_start(c + TWO_N, b)

            @pl.when(c + NBUF < NCD)
            def _():
                idx_wait(is2)
                g_start(is2, gs)

    plsc.subcore_barrier()
    pltpu.sync_copy(acc.at[pl.ds(sid * STRIPE, STRIPE)],
                    out_hbm.at[cid, pl.ds(sid * STRIPE, STRIPE)])


def _spmm_pass(t, ei, zeros128):
    k = pl.kernel(
        _spmm_body,
        out_type=jax.ShapeDtypeStruct((NCORE, NROWS, H), jnp.float32),
        mesh=plsc.VectorSubcoreMesh(
            core_axis_name="c", subcore_axis_name="s",
            num_cores=NCORE, num_subcores=NSUB),
        scratch_types=[
            pltpu.VMEM((TWO_N, 2, B), jnp.int32),
            pltpu.VMEM((NBUF, B, H), jnp.float32),
            pltpu.SemaphoreType.DMA((TWO_N,)),
            pltpu.SemaphoreType.DMA((NBUF,)),
            pltpu.VMEM_SHARED((NROWS, H), jnp.float32),
        ],
    )
    return k(t, ei, zeros128)


def _tc_first_body(x_ref, d0_ref, d1_ref, w_ref, t_ref, dinv_ref):
    deg = d0_ref[...] + d1_ref[...] + 1.0
    dinv = lax.rsqrt(deg)
    dinv_ref[...] = dinv[:, 0:16]
    t_ref[...] = dinv[:, 0:1] * jnp.dot(
        x_ref[...], w_ref[...], preferred_element_type=jnp.float32)


def _tc_first(x, degp, w):
    return pl.pallas_call(
        _tc_first_body,
        grid=(GRID,),
        in_specs=[
            pl.BlockSpec((BLK, D), lambda i: (i, 0)),
            pl.BlockSpec((BLK, WD), lambda i: (i, 0)),
            pl.BlockSpec((BLK, WD), lambda i: (i, 0)),
            pl.BlockSpec((D, H), lambda i: (0, 0)),
        ],
        out_specs=[
            pl.BlockSpec((BLK, H), lambda i: (i, 0)),
            pl.BlockSpec((BLK, 16), lambda i: (i, 0)),
        ],
        out_shape=[
            jax.ShapeDtypeStruct((NROWS, H), jnp.float32),
            jax.ShapeDtypeStruct((NROWS, 16), jnp.float32),
        ],
    )(x, degp[0], degp[1], w)


def _tc_mid_body(p0_ref, p1_ref, t_ref, dinv_ref, b_ref, w_ref, o_ref):
    dinv = dinv_ref[:, 0:1]
    s = p0_ref[...] + p1_ref[...] + t_ref[...]
    h = jnp.maximum(dinv * s + b_ref[...], 0.0)
    o_ref[...] = dinv * jnp.dot(
        h, w_ref[...], preferred_element_type=jnp.float32)


def _tc_mid(p, t, dinv, bias, w):
    return pl.pallas_call(
        _tc_mid_body,
        grid=(GRID,),
        in_specs=[
            pl.BlockSpec((BLK, H), lambda i: (i, 0)),
            pl.BlockSpec((BLK, H), lambda i: (i, 0)),
            pl.BlockSpec((BLK, H), lambda i: (i, 0)),
            pl.BlockSpec((BLK, 16), lambda i: (i, 0)),
            pl.BlockSpec((1, H), lambda i: (0, 0)),
            pl.BlockSpec((H, H), lambda i: (0, 0)),
        ],
        out_specs=pl.BlockSpec((BLK, H), lambda i: (i, 0)),
        out_shape=jax.ShapeDtypeStruct((NROWS, H), jnp.float32),
    )(p[0], p[1], t, dinv, bias, w)


def _tc_final_body(p0_ref, p1_ref, t_ref, dinv_ref, b_ref, batch_ref,
                   wl_ref, bl_ref, o_ref, pool_acc, cnt_acc):
    i = pl.program_id(0)

    @pl.when(i == 0)
    def _():
        pool_acc[...] = jnp.zeros_like(pool_acc)
        cnt_acc[...] = jnp.zeros_like(cnt_acc)

    dinv = dinv_ref[:, 0:1]
    h = dinv * (p0_ref[...] + p1_ref[...] + t_ref[...]) + b_ref[...]
    gid = batch_ref[...]
    onehot = jnp.where(
        gid == lax.broadcasted_iota(jnp.int32, (BLK, G), 1), 1.0, 0.0)
    pool_acc[...] += lax.dot_general(
        onehot, h, (((0,), (0,)), ((), ())),
        preferred_element_type=jnp.float32)
    cnt_acc[...] += lax.dot_general(
        onehot, jnp.ones((BLK, H), jnp.float32), (((0,), (0,)), ((), ())),
        preferred_element_type=jnp.float32)

    @pl.when(i == GRID - 1)
    def _():
        pooled = pool_acc[...] / jnp.maximum(cnt_acc[...], 1.0)
        o_ref[...] = jnp.dot(
            pooled, wl_ref[...], preferred_element_type=jnp.float32
        ) + bl_ref[...]


def _tc_final(p, t, dinv, bias, batch2d, wl, bl):
    return pl.pallas_call(
        _tc_final_body,
        grid=(GRID,),
        in_specs=[
            pl.BlockSpec((BLK, H), lambda i: (i, 0)),
            pl.BlockSpec((BLK, H), lambda i: (i, 0)),
            pl.BlockSpec((BLK, H), lambda i: (i, 0)),
            pl.BlockSpec((BLK, 16), lambda i: (i, 0)),
            pl.BlockSpec((1, H), lambda i: (0, 0)),
            pl.BlockSpec((BLK, 1), lambda i: (i, 0)),
            pl.BlockSpec((H, C), lambda i: (0, 0)),
            pl.BlockSpec((1, C), lambda i: (0, 0)),
        ],
        out_specs=pl.BlockSpec((G, C), lambda i: (0, 0)),
        out_shape=jax.ShapeDtypeStruct((G, C), jnp.float32),
        scratch_shapes=[
            pltpu.VMEM((G, H), jnp.float32),
            pltpu.VMEM((G, H), jnp.float32),
        ],
    )(p[0], p[1], t, dinv, bias, batch2d, wl, bl)


@jax.jit
def kernel(x, edge_index, batch, W1, b1, W2, b2, W3, b3, Wl, bl):
    pad_e = EPAD - E
    pad_ids = (jnp.arange(pad_e, dtype=jnp.int32) % (NROWS - N)) + N
    src = jnp.concatenate([edge_index[0], pad_ids]).reshape(-1, B)
    dst = jnp.concatenate([edge_index[1], pad_ids]).reshape(-1, B)
    nw = NCORE * NSUB
    ei = jnp.stack([src.reshape(nw, NCD, B),
                    dst.reshape(nw, NCD, B)], axis=2).reshape(-1, B)
    xp = jnp.concatenate(
        [x, jnp.zeros((NROWS - N, D), jnp.float32)], axis=0)
    batch2d = jnp.concatenate(
        [batch, jnp.full((NROWS - N,), G, jnp.int32)]).reshape(NROWS, 1)
    ones_deg = jnp.ones((B, WD), jnp.float32)
    zeros_deg = jnp.zeros((STRIPE, WD), jnp.float32)
    zeros128 = jnp.zeros((STRIPE, H), jnp.float32)
    b1r = b1.reshape(1, H)
    b2r = b2.reshape(1, H)
    b3r = b3.reshape(1, H)
    blr = bl.reshape(1, C)

    degp = _deg_pass(dst, ones_deg, zeros_deg)
    t1, dinv = _tc_first(xp, degp, W1)
    p1 = _spmm_pass(t1, ei, zeros128)
    t2 = _tc_mid(p1, t1, dinv, b1r, W2)
    p2 = _spmm_pass(t2, ei, zeros128)
    t3 = _tc_mid(p2, t2, dinv, b2r, W3)
    p3 = _spmm_pass(t3, ei, zeros128)
    return _tc_final(p3, t3, dinv, b3r, batch2d, Wl, blr)
